# Initial kernel scaffold; baseline (speedup 1.0000x reference)
import jax
import jax.numpy as jnp
from jax.experimental import pallas as pl


def kernel(length, table):
    raise NotImplementedError("write your pallas kernel here")


# speedup vs baseline: 9.7836x; 9.7836x over previous
import functools

import jax
import jax.numpy as jnp
from jax import lax
from jax.experimental import pallas as pl
from jax.experimental.pallas import tpu as pltpu
from jax.experimental.pallas import tpu_sc as plsc

_MAX_REL = 128
_EMB = 16
_VOCAB = 2 * _MAX_REL + 1
_LEN = 2048
_SLEN = 2 * _LEN
_NC = 2
_NS = 16
_NW = _NC * _NS
_ROWS_PER_W = _LEN // _NW

_MID_LO = _LEN - _MAX_REL - 1
_MID_HI = _MID_LO + _VOCAB


@functools.partial(
    pl.kernel,
    out_type=jax.ShapeDtypeStruct((_LEN, _LEN, _EMB), jnp.float32),
    mesh=plsc.VectorSubcoreMesh(core_axis_name="c", subcore_axis_name="s"),
    scratch_types=[
        pltpu.VMEM((_VOCAB, _EMB), jnp.float32),
        pltpu.VMEM((_SLEN, _EMB), jnp.float32),
    ],
    compiler_params=pltpu.CompilerParams(use_tc_tiling_on_sc=False),
)
def _rpe_sc(table_hbm, out_hbm, table_v, strip_v):
    wid = lax.axis_index("s") * _NC + lax.axis_index("c")

    pltpu.sync_copy(table_hbm, table_v)

    v_hi = table_v[_VOCAB - 1, :]
    v_lo = table_v[0, :]

    def fill_hi(m, carry):
        strip_v[m, :] = v_hi
        return carry

    def fill_mid(m, carry):
        strip_v[m, :] = table_v[_MID_HI - 1 - m, :]
        return carry

    def fill_lo(m, carry):
        strip_v[m, :] = v_lo
        return carry

    lax.fori_loop(0, _MID_LO, fill_hi, 0, unroll=4)
    lax.fori_loop(_MID_LO, _MID_HI, fill_mid, 0, unroll=4)
    lax.fori_loop(_MID_HI, _SLEN, fill_lo, 0, unroll=4)

    base = wid * _ROWS_PER_W

    def copy_row(t, carry):
        i = base + t
        pltpu.sync_copy(
            strip_v.at[pl.ds(_LEN - 1 - i, _LEN), :],
            out_hbm.at[i],
        )
        return carry

    lax.fori_loop(0, _ROWS_PER_W, copy_row, 0)


def kernel(length, table):
    del length
    return _rpe_sc(table)


# speedup vs baseline: 9.7872x; 1.0004x over previous
import functools

import jax
import jax.numpy as jnp
from jax import lax
from jax.experimental import pallas as pl
from jax.experimental.pallas import tpu as pltpu
from jax.experimental.pallas import tpu_sc as plsc

_MAX_REL = 128
_EMB = 16
_VOCAB = 2 * _MAX_REL + 1
_LEN = 2048
_SLEN = 2 * _LEN
_ROW_W = _LEN * _EMB
_NC = 2
_NS = 16
_NW = _NC * _NS
_ROWS_PER_W = _LEN // _NW

_MID_LO = _LEN - _MAX_REL - 1
_MID_HI = _MID_LO + _VOCAB


@functools.partial(
    pl.kernel,
    out_type=jax.ShapeDtypeStruct((_LEN * _LEN * _EMB,), jnp.float32),
    mesh=plsc.VectorSubcoreMesh(core_axis_name="c", subcore_axis_name="s"),
    scratch_types=[
        pltpu.VMEM((_VOCAB, _EMB), jnp.float32),
        pltpu.VMEM((_SLEN * _EMB,), jnp.float32),
    ],
    compiler_params=pltpu.CompilerParams(use_tc_tiling_on_sc=False),
)
def _rpe_sc(table_hbm, out_hbm, table_v, strip_v):
    wid = lax.axis_index("s") * _NC + lax.axis_index("c")

    pltpu.sync_copy(table_hbm, table_v)

    v_hi = table_v[_VOCAB - 1, :]
    v_lo = table_v[0, :]

    def fill_hi(m, carry):
        strip_v[pl.ds(m * _EMB, _EMB)] = v_hi
        return carry

    def fill_mid(m, carry):
        strip_v[pl.ds(m * _EMB, _EMB)] = table_v[_MID_HI - 1 - m, :]
        return carry

    def fill_lo(m, carry):
        strip_v[pl.ds(m * _EMB, _EMB)] = v_lo
        return carry

    lax.fori_loop(0, _MID_LO, fill_hi, 0, unroll=4)
    lax.fori_loop(_MID_LO, _MID_HI, fill_mid, 0, unroll=4)
    lax.fori_loop(_MID_HI, _SLEN, fill_lo, 0, unroll=4)

    base = wid * _ROWS_PER_W

    def copy_row(t, carry):
        i = base + t
        pltpu.sync_copy(
            strip_v.at[pl.ds((_LEN - 1 - i) * _EMB, _ROW_W)],
            out_hbm.at[pl.ds(i * _ROW_W, _ROW_W)],
        )
        return carry

    lax.fori_loop(0, _ROWS_PER_W, copy_row, 0)


def kernel(length, table):
    del length
    return _rpe_sc(table).reshape(_LEN, _LEN, _EMB)


# speedup vs baseline: 134.1450x; 13.7062x over previous
import functools

import jax
import jax.numpy as jnp
from jax import lax
from jax.experimental import pallas as pl
from jax.experimental.pallas import tpu as pltpu
from jax.experimental.pallas import tpu_sc as plsc

_MAX_REL = 128
_EMB = 16
_VOCAB = 2 * _MAX_REL + 1
_LEN = 2048
_SLEN = 2 * _LEN
_NC = 2
_NS = 16
_NW = _NC * _NS
_ROWS_PER_W = _LEN // _NW


@functools.partial(
    pl.kernel,
    out_type=jax.ShapeDtypeStruct((_LEN, 2, 16, 8, 128), jnp.float32),
    mesh=plsc.VectorSubcoreMesh(core_axis_name="c", subcore_axis_name="s"),
    scratch_types=[
        pltpu.VMEM((_VOCAB, _EMB), jnp.float32),
        pltpu.VMEM((2, 8, _SLEN), jnp.float32),
        pltpu.SemaphoreType.DMA,
    ],
    compiler_params=pltpu.CompilerParams(use_tc_tiling_on_sc=False, needs_layout_passes=False),
)
def _rpe_sc(table_hbm, out_hbm, table_v, sarr_v, sem):
    wid = lax.axis_index("s") * _NC + lax.axis_index("c")
    r = wid % 8
    q = wid // 8
    phi = (7 - r) % 8

    pltpu.sync_copy(table_hbm, table_v)

    lane = lax.iota(jnp.int32, 16)

    def build(u, carry):
        e = u // (_SLEN // 16)
        m0 = (u % (_SLEN // 16)) * 16
        row = jnp.clip(2047 - phi - m0 - lane, -_MAX_REL, _MAX_REL) + _MAX_REL
        v = plsc.load_gather(table_v, [row, jnp.full((16,), e, jnp.int32)])
        sarr_v[e // 8, e % 8, pl.ds(m0, 16)] = v
        return carry

    lax.fori_loop(0, _EMB * (_SLEN // 16), build, 0, unroll=4)

    def copy_plane(k, carry):
        i = r + 8 * (4 * k + q)
        off = pl.multiple_of((2047 - i) - phi, 8)
        descs = [
            pltpu.async_copy(
                sarr_v.at[:, :, pl.ds(pl.multiple_of(off + 128 * t, 8), 128)],
                out_hbm.at[i, :, t],
                sem,
            )
            for t in range(16)
        ]
        for d in descs:
            d.wait()
        return carry

    lax.fori_loop(0, _ROWS_PER_W, copy_plane, 0)


def kernel(length, table):
    del length
    out = _rpe_sc(table)
    out = out.transpose(0, 2, 4, 1, 3)
    return out.reshape(_LEN, _LEN, _EMB)


# speedup vs baseline: 142.2992x; 1.0608x over previous
import functools

import jax
import jax.numpy as jnp
from jax import lax
from jax.experimental import pallas as pl
from jax.experimental.pallas import tpu as pltpu
from jax.experimental.pallas import tpu_sc as plsc

_MAX_REL = 128
_EMB = 16
_VOCAB = 2 * _MAX_REL + 1
_LEN = 2048
_SLEN = 2 * _LEN
_NC = 2
_NS = 16
_NW = _NC * _NS
_ROWS_PER_W = _LEN // _NW


@functools.partial(
    pl.kernel,
    out_type=jax.ShapeDtypeStruct((_LEN, 2, 16, 8, 128), jnp.float32),
    mesh=plsc.VectorSubcoreMesh(core_axis_name="c", subcore_axis_name="s"),
    scratch_types=[
        pltpu.VMEM((_VOCAB, _EMB), jnp.float32),
        pltpu.VMEM((2, 8, _SLEN), jnp.float32),
        pltpu.SemaphoreType.DMA,
    ],
    compiler_params=pltpu.CompilerParams(use_tc_tiling_on_sc=False, needs_layout_passes=False),
)
def _rpe_sc(table_hbm, out_hbm, table_v, sarr_v, sem):
    wid = lax.axis_index("s") * _NC + lax.axis_index("c")
    r = wid % 8
    q = wid // 8
    phi = (7 - r) % 8

    pltpu.sync_copy(table_hbm, table_v)

    lane = lax.iota(jnp.int32, 16)

    off_min = 1536 - 512 * q

    def build_e(e, carry):
        ef = jnp.full((16,), e, jnp.int32)
        d0 = e // 8
        d1 = e % 8

        def build_chunk(c, carry2):
            m0 = off_min + 16 * c
            row = jnp.clip(2047 - phi - m0 - lane, -_MAX_REL, _MAX_REL) + _MAX_REL
            v = plsc.load_gather(table_v, [row, ef])
            sarr_v[d0, d1, pl.ds(m0, 16)] = v
            return carry2

        lax.fori_loop(0, 160, build_chunk, 0, unroll=4)
        return carry

    lax.fori_loop(0, _EMB, build_e, 0)

    def copy_plane(k, carry):
        i = r + 512 * q + 8 * k
        off = pl.multiple_of((2047 - i) - phi, 8)
        descs = [
            pltpu.async_copy(
                sarr_v.at[:, :, pl.ds(pl.multiple_of(off + 128 * t, 8), 128)],
                out_hbm.at[i, :, t],
                sem,
            )
            for t in range(16)
        ]
        for d in descs:
            d.wait()
        return carry

    lax.fori_loop(0, _ROWS_PER_W, copy_plane, 0)


def kernel(length, table):
    del length
    out = _rpe_sc(table)
    out = out.transpose(0, 2, 4, 1, 3)
    return out.reshape(_LEN, _LEN, _EMB)
